# baseline (speedup 1.0000x reference)
import jax
import jax.numpy as jnp
from jax import lax
from jax.experimental import pallas as pl
from jax.experimental.pallas import tpu as pltpu

_BN_EPS = 1e-5


def _whole(shape):
    n = len(shape)
    return pl.BlockSpec(shape, lambda b, _n=n: (0,) * _n)


def _interleave_rows(a, b):
    h, w, c = a.shape
    t = jnp.concatenate([a[:, None], b[:, None]], axis=1)
    return t.reshape(2 * h, w, c)


def _stage1_kernel(x_ref, s1_ref, t1_ref, wee_ref, weo_ref, woe_ref, woo_ref,
                   b3a_ref, h_ref, stats_ref):
    _, H, W, C = x_ref.shape

    x2d = x_ref[0].reshape(H * W, C)
    a2d = jnp.maximum(x2d * s1_ref[...] + t1_ref[...], 0.0)
    a = a2d.astype(jnp.bfloat16).reshape(H, W, C)

    zc = jnp.zeros((H, 1, C), jnp.bfloat16)
    aw = jnp.concatenate([zc, a, zc], axis=1)
    zr = jnp.zeros((1, W + 2, C), jnp.bfloat16)
    ap = jnp.concatenate([zr, aw, zr], axis=0)

    def parity(pr, pc, w_ref):
        taps = [ap[pr + ty:pr + ty + H, pc + tx:pc + tx + W, :]
                for ty in range(2) for tx in range(2)]
        im = jnp.concatenate(taps, axis=-1).reshape(H * W, 4 * C)
        return (jnp.dot(im, w_ref[...], preferred_element_type=jnp.float32)
                + b3a_ref[...])

    o00 = parity(0, 0, wee_ref)
    o01 = parity(0, 1, weo_ref)
    o10 = parity(1, 0, woe_ref)
    o11 = parity(1, 1, woo_ref)

    s = (jnp.sum(o00, axis=0, keepdims=True) + jnp.sum(o01, axis=0, keepdims=True)
         + jnp.sum(o10, axis=0, keepdims=True) + jnp.sum(o11, axis=0, keepdims=True))
    ss = (jnp.sum(o00 * o00, axis=0, keepdims=True)
          + jnp.sum(o01 * o01, axis=0, keepdims=True)
          + jnp.sum(o10 * o10, axis=0, keepdims=True)
          + jnp.sum(o11 * o11, axis=0, keepdims=True))
    stats_ref[0, 0:1, :] = s
    stats_ref[0, 1:2, :] = ss

    he = _interleave_rows(o00.astype(jnp.bfloat16).reshape(H, W, C),
                          o10.astype(jnp.bfloat16).reshape(H, W, C))
    ho = _interleave_rows(o01.astype(jnp.bfloat16).reshape(H, W, C),
                          o11.astype(jnp.bfloat16).reshape(H, W, C))
    h_ref[0] = jnp.concatenate([he, ho], axis=-1)


def _stage2_kernel(h_ref, x_ref, s2_ref, t2_ref, w3b_ref, b3b_ref,
                   w1_ref, b1_ref, o_ref):
    _, H2, W, C2 = h_ref.shape
    C = C2 // 2
    H = H2 // 2
    Cout = o_ref.shape[-1] // 2

    h = h_ref[0].astype(jnp.float32)
    s2 = s2_ref[...]
    t2 = t2_ref[...]
    ae = jnp.maximum(h[:, :, :C] * s2 + t2, 0.0).astype(jnp.bfloat16)
    ao = jnp.maximum(h[:, :, C:] * s2 + t2, 0.0).astype(jnp.bfloat16)

    zc = jnp.zeros((H2, 1, C), jnp.bfloat16)
    ao_m1 = jnp.concatenate([zc, ao[:, :W - 1]], axis=1)
    ae_p1 = jnp.concatenate([ae[:, 1:], zc], axis=1)
    zr = jnp.zeros((1, W, C), jnp.bfloat16)

    def hp(t):
        return jnp.concatenate([zr, t, zr], axis=0)

    pae, pao, pao_m1, pae_p1 = hp(ae), hp(ao), hp(ao_m1), hp(ae_p1)

    w3b = w3b_ref[...]
    ev = [src[dy:dy + H2] for dy in range(3) for src in (pao_m1, pae, pao)]
    even_im = jnp.concatenate(ev, axis=-1).reshape(H2 * W, 9 * C)
    ye = jnp.dot(even_im, w3b, preferred_element_type=jnp.float32) + b3b_ref[...]
    od = [src[dy:dy + H2] for dy in range(3) for src in (pae, pao, pae_p1)]
    odd_im = jnp.concatenate(od, axis=-1).reshape(H2 * W, 9 * C)
    yo = jnp.dot(odd_im, w3b, preferred_element_type=jnp.float32) + b3b_ref[...]

    x2d = x_ref[0].reshape(H * W, C).astype(jnp.bfloat16)
    sc = (jnp.dot(x2d, w1_ref[...], preferred_element_type=jnp.float32)
          + b1_ref[...]).reshape(H, W, Cout)
    sc2 = _interleave_rows(sc, sc)

    o_ref[0] = jnp.concatenate([ye.reshape(H2, W, Cout) + sc2,
                                yo.reshape(H2, W, Cout) + sc2], axis=-1)


def kernel(x, w1x1, b1x1, w3a_flat, b3a, w3b_flat, b3b, g1, be1, g2, be2):
    B, H, W, C = x.shape
    Cout = w1x1.shape[-1]
    H2, W2 = 2 * H, 2 * W
    x = x.astype(jnp.float32)

    m1 = jnp.mean(x, axis=(0, 1, 2))
    v1 = jnp.mean(x * x, axis=(0, 1, 2)) - m1 * m1
    inv1 = lax.rsqrt(v1 + _BN_EPS)
    scale1 = (g1 * inv1).reshape(1, C)
    shift1 = (be1 - m1 * g1 * inv1).reshape(1, C)

    w9 = w3a_flat.reshape(3, 3, C, C)
    groups = (((0,), (1, 2)), ((0, 1), (2,)))

    def comb(pr, pc):
        blocks = []
        for ty in range(2):
            for tx in range(2):
                blk = 0.0
                for dy in groups[pr][ty]:
                    for dx in groups[pc][tx]:
                        blk = blk + w9[dy, dx]
                blocks.append(blk)
        return jnp.concatenate(blocks, axis=0).astype(jnp.bfloat16)

    wee, weo, woe, woo = comb(0, 0), comb(0, 1), comb(1, 0), comb(1, 1)

    cparams = pltpu.CompilerParams(
        dimension_semantics=("parallel",),
        vmem_limit_bytes=64 * 1024 * 1024,
    )

    h, stats = pl.pallas_call(
        _stage1_kernel,
        out_shape=(jax.ShapeDtypeStruct((B, H2, W, 2 * C), jnp.bfloat16),
                   jax.ShapeDtypeStruct((B, 2, C), jnp.float32)),
        grid=(B,),
        in_specs=[
            pl.BlockSpec((1, H, W, C), lambda b: (b, 0, 0, 0)),
            _whole((1, C)), _whole((1, C)),
            _whole((4 * C, C)), _whole((4 * C, C)),
            _whole((4 * C, C)), _whole((4 * C, C)),
            _whole((1, C)),
        ],
        out_specs=(
            pl.BlockSpec((1, H2, W, 2 * C), lambda b: (b, 0, 0, 0)),
            pl.BlockSpec((1, 2, C), lambda b: (b, 0, 0)),
        ),
        compiler_params=cparams,
    )(x, scale1, shift1, wee, weo, woe, woo, b3a)

    n2 = jnp.float32(B * H2 * W2)
    tot = jnp.sum(stats, axis=0)
    m2 = tot[0] / n2
    v2 = tot[1] / n2 - m2 * m2
    inv2 = lax.rsqrt(v2 + _BN_EPS)
    scale2 = (g2 * inv2).reshape(1, C)
    shift2 = (be2 - m2 * g2 * inv2).reshape(1, C)

    out = pl.pallas_call(
        _stage2_kernel,
        out_shape=jax.ShapeDtypeStruct((B, H2, W, 2 * Cout), jnp.float32),
        grid=(B,),
        in_specs=[
            pl.BlockSpec((1, H2, W, 2 * C), lambda b: (b, 0, 0, 0)),
            pl.BlockSpec((1, H, W, C), lambda b: (b, 0, 0, 0)),
            _whole((1, C)), _whole((1, C)),
            _whole((9 * C, Cout)), _whole((1, Cout)),
            _whole((C, Cout)), _whole((1, Cout)),
        ],
        out_specs=pl.BlockSpec((1, H2, W, 2 * Cout), lambda b: (b, 0, 0, 0)),
        compiler_params=cparams,
    )(h, x, scale2, shift2, w3b_flat.astype(jnp.bfloat16), b3b,
      w1x1.astype(jnp.bfloat16), b1x1)

    return out.reshape(B, H2, W2, Cout)


# speedup vs baseline: 1.5571x; 1.5571x over previous
import jax
import jax.numpy as jnp
from jax import lax
from jax.experimental import pallas as pl
from jax.experimental.pallas import tpu as pltpu

_BN_EPS = 1e-5


def _whole(shape):
    n = len(shape)
    return pl.BlockSpec(shape, lambda b, _n=n: (0,) * _n)


def _interleave_rows(a, b):
    n, h, w, c = a.shape
    t = jnp.concatenate([a[:, :, None], b[:, :, None]], axis=2)
    return t.reshape(n, 2 * h, w, c)


def _stage1_kernel(x_ref, s1_ref, t1_ref, wee_ref, weo_ref, woe_ref, woo_ref,
                   b3a_ref, h_ref, stats_ref):
    NB, H, W, C = x_ref.shape

    x2d = x_ref[...].reshape(NB * H * W, C)
    a2d = jnp.maximum(x2d * s1_ref[...] + t1_ref[...], 0.0)
    a = a2d.astype(jnp.bfloat16).reshape(NB, H, W, C)

    zc = jnp.zeros((NB, H, 1, C), jnp.bfloat16)
    aw = jnp.concatenate([zc, a, zc], axis=2)
    zr = jnp.zeros((NB, 1, W + 2, C), jnp.bfloat16)
    ap = jnp.concatenate([zr, aw, zr], axis=1)

    def parity(pr, pc, w_ref):
        taps = [ap[:, pr + ty:pr + ty + H, pc + tx:pc + tx + W, :]
                for ty in range(2) for tx in range(2)]
        im = jnp.concatenate(taps, axis=-1).reshape(NB * H * W, 4 * C)
        return (jnp.dot(im, w_ref[...], preferred_element_type=jnp.float32)
                + b3a_ref[...])

    o00 = parity(0, 0, wee_ref)
    o01 = parity(0, 1, weo_ref)
    o10 = parity(1, 0, woe_ref)
    o11 = parity(1, 1, woo_ref)

    s = (jnp.sum(o00, axis=0, keepdims=True) + jnp.sum(o01, axis=0, keepdims=True)
         + jnp.sum(o10, axis=0, keepdims=True) + jnp.sum(o11, axis=0, keepdims=True))
    ss = (jnp.sum(o00 * o00, axis=0, keepdims=True)
          + jnp.sum(o01 * o01, axis=0, keepdims=True)
          + jnp.sum(o10 * o10, axis=0, keepdims=True)
          + jnp.sum(o11 * o11, axis=0, keepdims=True))
    stats_ref[0, 0:1, :] = s
    stats_ref[0, 1:2, :] = ss

    he = _interleave_rows(o00.astype(jnp.bfloat16).reshape(NB, H, W, C),
                          o10.astype(jnp.bfloat16).reshape(NB, H, W, C))
    ho = _interleave_rows(o01.astype(jnp.bfloat16).reshape(NB, H, W, C),
                          o11.astype(jnp.bfloat16).reshape(NB, H, W, C))
    h_ref[...] = jnp.concatenate([he, ho], axis=-1)


def _stage2_kernel(h_ref, x_ref, s2_ref, t2_ref, w3b_ref,
                   w1_ref, b1_ref, sel_ref, o_ref):
    NB, H2, W, C2 = h_ref.shape
    C = C2 // 2
    H = H2 // 2
    W2 = 2 * W
    Cout = o_ref.shape[-1]

    h = h_ref[...].astype(jnp.float32)
    s2 = s2_ref[...]
    t2 = t2_ref[...]
    ae = jnp.maximum(h[..., :C] * s2 + t2, 0.0).astype(jnp.bfloat16)
    ao = jnp.maximum(h[..., C:] * s2 + t2, 0.0).astype(jnp.bfloat16)

    zc = jnp.zeros((NB, H2, 1, C), jnp.bfloat16)
    ao_m1 = jnp.concatenate([zc, ao[:, :, :W - 1]], axis=2)
    ae_p1 = jnp.concatenate([ae[:, :, 1:], zc], axis=2)
    zr = jnp.zeros((NB, 1, W, C), jnp.bfloat16)

    def hp(t):
        return jnp.concatenate([zr, t, zr], axis=1)

    pae, pao, pao_m1, pae_p1 = hp(ae), hp(ao), hp(ao_m1), hp(ae_p1)

    w3b = w3b_ref[...]
    ev = [src[:, dy:dy + H2] for dy in range(3) for src in (pao_m1, pae, pao)]
    even_im = jnp.concatenate(ev, axis=-1).reshape(NB * H2 * W, 9 * C)
    ye = jnp.dot(even_im, w3b, preferred_element_type=jnp.float32)
    od = [src[:, dy:dy + H2] for dy in range(3) for src in (pae, pao, pae_p1)]
    odd_im = jnp.concatenate(od, axis=-1).reshape(NB * H2 * W, 9 * C)
    yo = jnp.dot(odd_im, w3b, preferred_element_type=jnp.float32)

    x2d = x_ref[...].reshape(NB * H * W, C).astype(jnp.bfloat16)
    sc = (jnp.dot(x2d, w1_ref[...], preferred_element_type=jnp.float32)
          + b1_ref[...]).reshape(NB, H, W, Cout)
    sc2 = _interleave_rows(sc, sc)

    n = NB * H2
    st = jnp.concatenate([ye.reshape(n, W, Cout), yo.reshape(n, W, Cout),
                          sc2.reshape(n, W, Cout)], axis=1)
    sel = jnp.broadcast_to(sel_ref[...], (n, W2, 3 * W))
    out = jnp.einsum("hxw,hwc->hxc", sel, st,
                     preferred_element_type=jnp.float32)
    o_ref[...] = out.reshape(NB, H2, W2, Cout)


def kernel(x, w1x1, b1x1, w3a_flat, b3a, w3b_flat, b3b, g1, be1, g2, be2):
    B, H, W, C = x.shape
    Cout = w1x1.shape[-1]
    H2, W2 = 2 * H, 2 * W
    x = x.astype(jnp.float32)
    NB1 = 8 if B % 8 == 0 else (2 if B % 2 == 0 else 1)
    NB = 4 if B % 4 == 0 else (2 if B % 2 == 0 else 1)

    m1 = jnp.mean(x, axis=(0, 1, 2))
    v1 = jnp.mean(x * x, axis=(0, 1, 2)) - m1 * m1
    inv1 = lax.rsqrt(v1 + _BN_EPS)
    scale1 = (g1 * inv1).reshape(1, C)
    shift1 = (be1 - m1 * g1 * inv1).reshape(1, C)

    w9 = w3a_flat.reshape(3, 3, C, C)
    groups = (((0,), (1, 2)), ((0, 1), (2,)))

    def comb(pr, pc):
        blocks = []
        for ty in range(2):
            for tx in range(2):
                blk = 0.0
                for dy in groups[pr][ty]:
                    for dx in groups[pc][tx]:
                        blk = blk + w9[dy, dx]
                blocks.append(blk)
        return jnp.concatenate(blocks, axis=0).astype(jnp.bfloat16)

    wee, weo, woe, woo = comb(0, 0), comb(0, 1), comb(1, 0), comb(1, 1)

    cparams = pltpu.CompilerParams(
        dimension_semantics=("parallel",),
        vmem_limit_bytes=100 * 1024 * 1024,
    )

    h, stats = pl.pallas_call(
        _stage1_kernel,
        out_shape=(jax.ShapeDtypeStruct((B, H2, W, 2 * C), jnp.bfloat16),
                   jax.ShapeDtypeStruct((B // NB1, 2, C), jnp.float32)),
        grid=(B // NB1,),
        in_specs=[
            pl.BlockSpec((NB1, H, W, C), lambda b: (b, 0, 0, 0)),
            _whole((1, C)), _whole((1, C)),
            _whole((4 * C, C)), _whole((4 * C, C)),
            _whole((4 * C, C)), _whole((4 * C, C)),
            _whole((1, C)),
        ],
        out_specs=(
            pl.BlockSpec((NB1, H2, W, 2 * C), lambda b: (b, 0, 0, 0)),
            pl.BlockSpec((1, 2, C), lambda b: (b, 0, 0)),
        ),
        compiler_params=cparams,
    )(x, scale1, shift1, wee, weo, woe, woo, b3a)

    n2 = jnp.float32(B * H2 * W2)
    tot = jnp.sum(stats, axis=0)
    m2 = tot[0] / n2
    v2 = tot[1] / n2 - m2 * m2
    inv2 = lax.rsqrt(v2 + _BN_EPS)
    scale2 = (g2 * inv2).reshape(1, C)
    shift2 = (be2 - m2 * g2 * inv2).reshape(1, C)

    xi = lax.broadcasted_iota(jnp.int32, (1, W2, 3 * W), 1)
    wi = lax.broadcasted_iota(jnp.int32, (1, W2, 3 * W), 2)
    sel = ((xi == 2 * wi) | (xi == 2 * wi - 2 * W + 1)
           | (xi == 2 * wi - 4 * W) | (xi == 2 * wi - 4 * W + 1)
           ).astype(jnp.float32)

    out = pl.pallas_call(
        _stage2_kernel,
        out_shape=jax.ShapeDtypeStruct((B, H2, W2, Cout), jnp.float32),
        grid=(B // NB,),
        in_specs=[
            pl.BlockSpec((NB, H2, W, 2 * C), lambda b: (b, 0, 0, 0)),
            pl.BlockSpec((NB, H, W, C), lambda b: (b, 0, 0, 0)),
            _whole((1, C)), _whole((1, C)),
            _whole((9 * C, Cout)),
            _whole((C, Cout)), _whole((1, Cout)),
            _whole((1, W2, 3 * W)),
        ],
        out_specs=pl.BlockSpec((NB, H2, W2, Cout), lambda b: (b, 0, 0, 0)),
        compiler_params=cparams,
    )(h, x, scale2, shift2, w3b_flat.astype(jnp.bfloat16),
      w1x1.astype(jnp.bfloat16), b1x1 + b3b, sel)
    return out


# speedup vs baseline: 1.5746x; 1.0113x over previous
import jax
import jax.numpy as jnp
from jax import lax
from jax.experimental import pallas as pl
from jax.experimental.pallas import tpu as pltpu

_BN_EPS = 1e-5


def _whole(shape):
    n = len(shape)
    return pl.BlockSpec(shape, lambda b, _n=n: (0,) * _n)


def _interleave_rows(a, b):
    n, h, w, c = a.shape
    t = jnp.concatenate([a[:, :, None], b[:, :, None]], axis=2)
    return t.reshape(n, 2 * h, w, c)


def _stage1_kernel(x_ref, s1_ref, t1_ref, wee_ref, weo_ref, woe_ref, woo_ref,
                   b3a_ref, h_ref, stats_ref):
    NB, H, W, C = x_ref.shape

    x2d = x_ref[...].reshape(NB * H * W, C)
    a2d = jnp.maximum(x2d * s1_ref[...] + t1_ref[...], 0.0)
    a = a2d.astype(jnp.bfloat16).reshape(NB, H, W, C)

    zc = jnp.zeros((NB, H, 1, C), jnp.bfloat16)
    aw = jnp.concatenate([zc, a, zc], axis=2)
    zr = jnp.zeros((NB, 1, W + 2, C), jnp.bfloat16)
    ap = jnp.concatenate([zr, aw, zr], axis=1)

    def parity(pr, pc, w_ref):
        taps = [ap[:, pr + ty:pr + ty + H, pc + tx:pc + tx + W, :]
                for ty in range(2) for tx in range(2)]
        im = jnp.concatenate(taps, axis=-1).reshape(NB * H * W, 4 * C)
        return (jnp.dot(im, w_ref[...], preferred_element_type=jnp.float32)
                + b3a_ref[...])

    o00 = parity(0, 0, wee_ref)
    o01 = parity(0, 1, weo_ref)
    o10 = parity(1, 0, woe_ref)
    o11 = parity(1, 1, woo_ref)

    s = (jnp.sum(o00, axis=0, keepdims=True) + jnp.sum(o01, axis=0, keepdims=True)
         + jnp.sum(o10, axis=0, keepdims=True) + jnp.sum(o11, axis=0, keepdims=True))
    ss = (jnp.sum(o00 * o00, axis=0, keepdims=True)
          + jnp.sum(o01 * o01, axis=0, keepdims=True)
          + jnp.sum(o10 * o10, axis=0, keepdims=True)
          + jnp.sum(o11 * o11, axis=0, keepdims=True))
    stats_ref[0, 0:1, :] = s
    stats_ref[0, 1:2, :] = ss

    he = _interleave_rows(o00.astype(jnp.bfloat16).reshape(NB, H, W, C),
                          o10.astype(jnp.bfloat16).reshape(NB, H, W, C))
    ho = _interleave_rows(o01.astype(jnp.bfloat16).reshape(NB, H, W, C),
                          o11.astype(jnp.bfloat16).reshape(NB, H, W, C))
    h_ref[...] = jnp.concatenate([he, ho], axis=-1)


def _stage2_kernel(h_ref, x_ref, s2_ref, t2_ref, w3b_ref,
                   w1_ref, b1_ref, sel_ref, o_ref):
    NB, H2, W, C2 = h_ref.shape
    C = C2 // 2
    H = H2 // 2
    W2 = 2 * W
    Cout = o_ref.shape[-1]

    h = h_ref[...].astype(jnp.float32)
    s2 = s2_ref[...]
    t2 = t2_ref[...]
    ae = jnp.maximum(h[..., :C] * s2 + t2, 0.0).astype(jnp.bfloat16)
    ao = jnp.maximum(h[..., C:] * s2 + t2, 0.0).astype(jnp.bfloat16)

    zc = jnp.zeros((NB, H2, 1, C), jnp.bfloat16)
    ao_m1 = jnp.concatenate([zc, ao[:, :, :W - 1]], axis=2)
    ae_p1 = jnp.concatenate([ae[:, :, 1:], zc], axis=2)
    zr = jnp.zeros((NB, 1, W, C), jnp.bfloat16)

    def hp(t):
        return jnp.concatenate([zr, t, zr], axis=1)

    pae, pao, pao_m1, pae_p1 = hp(ae), hp(ao), hp(ao_m1), hp(ae_p1)

    w3b = w3b_ref[...]
    ev = [src[:, dy:dy + H2] for dy in range(3) for src in (pao_m1, pae, pao)]
    even_im = jnp.concatenate(ev, axis=-1).reshape(NB * H2 * W, 9 * C)
    ye = jnp.dot(even_im, w3b, preferred_element_type=jnp.float32)
    od = [src[:, dy:dy + H2] for dy in range(3) for src in (pae, pao, pae_p1)]
    odd_im = jnp.concatenate(od, axis=-1).reshape(NB * H2 * W, 9 * C)
    yo = jnp.dot(odd_im, w3b, preferred_element_type=jnp.float32)

    x2d = x_ref[...].reshape(NB * H * W, C).astype(jnp.bfloat16)
    sc = (jnp.dot(x2d, w1_ref[...], preferred_element_type=jnp.float32)
          + b1_ref[...]).reshape(NB, H, W, Cout)
    sc2 = _interleave_rows(sc, sc)

    n = NB * H2
    st = jnp.concatenate([ye.reshape(n, W, Cout), yo.reshape(n, W, Cout),
                          sc2.reshape(n, W, Cout)], axis=1)
    sel = jnp.broadcast_to(sel_ref[...], (n, W2, 3 * W))
    out = jnp.einsum("hxw,hwc->hxc", sel, st,
                     preferred_element_type=jnp.float32)
    o_ref[...] = out.reshape(NB, H2, W2, Cout)


def kernel(x, w1x1, b1x1, w3a_flat, b3a, w3b_flat, b3b, g1, be1, g2, be2):
    B, H, W, C = x.shape
    Cout = w1x1.shape[-1]
    H2, W2 = 2 * H, 2 * W
    x = x.astype(jnp.float32)
    NB1 = 8 if B % 8 == 0 else (2 if B % 2 == 0 else 1)
    NB = 8 if B % 8 == 0 else (2 if B % 2 == 0 else 1)

    m1 = jnp.mean(x, axis=(0, 1, 2))
    v1 = jnp.mean(x * x, axis=(0, 1, 2)) - m1 * m1
    inv1 = lax.rsqrt(v1 + _BN_EPS)
    scale1 = (g1 * inv1).reshape(1, C)
    shift1 = (be1 - m1 * g1 * inv1).reshape(1, C)

    w9 = w3a_flat.reshape(3, 3, C, C)
    groups = (((0,), (1, 2)), ((0, 1), (2,)))

    def comb(pr, pc):
        blocks = []
        for ty in range(2):
            for tx in range(2):
                blk = 0.0
                for dy in groups[pr][ty]:
                    for dx in groups[pc][tx]:
                        blk = blk + w9[dy, dx]
                blocks.append(blk)
        return jnp.concatenate(blocks, axis=0).astype(jnp.bfloat16)

    wee, weo, woe, woo = comb(0, 0), comb(0, 1), comb(1, 0), comb(1, 1)

    cparams = pltpu.CompilerParams(
        dimension_semantics=("parallel",),
        vmem_limit_bytes=100 * 1024 * 1024,
    )

    h, stats = pl.pallas_call(
        _stage1_kernel,
        out_shape=(jax.ShapeDtypeStruct((B, H2, W, 2 * C), jnp.bfloat16),
                   jax.ShapeDtypeStruct((B // NB1, 2, C), jnp.float32)),
        grid=(B // NB1,),
        in_specs=[
            pl.BlockSpec((NB1, H, W, C), lambda b: (b, 0, 0, 0)),
            _whole((1, C)), _whole((1, C)),
            _whole((4 * C, C)), _whole((4 * C, C)),
            _whole((4 * C, C)), _whole((4 * C, C)),
            _whole((1, C)),
        ],
        out_specs=(
            pl.BlockSpec((NB1, H2, W, 2 * C), lambda b: (b, 0, 0, 0)),
            pl.BlockSpec((1, 2, C), lambda b: (b, 0, 0)),
        ),
        compiler_params=cparams,
    )(x, scale1, shift1, wee, weo, woe, woo, b3a)

    n2 = jnp.float32(B * H2 * W2)
    tot = jnp.sum(stats, axis=0)
    m2 = tot[0] / n2
    v2 = tot[1] / n2 - m2 * m2
    inv2 = lax.rsqrt(v2 + _BN_EPS)
    scale2 = (g2 * inv2).reshape(1, C)
    shift2 = (be2 - m2 * g2 * inv2).reshape(1, C)

    xi = lax.broadcasted_iota(jnp.int32, (1, W2, 3 * W), 1)
    wi = lax.broadcasted_iota(jnp.int32, (1, W2, 3 * W), 2)
    sel = ((xi == 2 * wi) | (xi == 2 * wi - 2 * W + 1)
           | (xi == 2 * wi - 4 * W) | (xi == 2 * wi - 4 * W + 1)
           ).astype(jnp.float32)

    out = pl.pallas_call(
        _stage2_kernel,
        out_shape=jax.ShapeDtypeStruct((B, H2, W2, Cout), jnp.float32),
        grid=(B // NB,),
        in_specs=[
            pl.BlockSpec((NB, H2, W, 2 * C), lambda b: (b, 0, 0, 0)),
            pl.BlockSpec((NB, H, W, C), lambda b: (b, 0, 0, 0)),
            _whole((1, C)), _whole((1, C)),
            _whole((9 * C, Cout)),
            _whole((C, Cout)), _whole((1, Cout)),
            _whole((1, W2, 3 * W)),
        ],
        out_specs=pl.BlockSpec((NB, H2, W2, Cout), lambda b: (b, 0, 0, 0)),
        compiler_params=cparams,
    )(h, x, scale2, shift2, w3b_flat.astype(jnp.bfloat16),
      w1x1.astype(jnp.bfloat16), b1x1 + b3b, sel)
    return out


# speedup vs baseline: 1.5756x; 1.0007x over previous
import jax
import jax.numpy as jnp
from jax import lax
from jax.experimental import pallas as pl
from jax.experimental.pallas import tpu as pltpu

_BN_EPS = 1e-5


def _whole(shape):
    n = len(shape)
    return pl.BlockSpec(shape, lambda b, _n=n: (0,) * _n)


def _interleave_rows(a, b):
    n, h, w, c = a.shape
    t = jnp.concatenate([a[:, :, None], b[:, :, None]], axis=2)
    return t.reshape(n, 2 * h, w, c)


def _stage1_kernel(x_ref, s1_ref, t1_ref, wee_ref, weo_ref, woe_ref, woo_ref,
                   b3a_ref, h_ref, stats_ref):
    NB, H, W, C = x_ref.shape

    x2d = x_ref[...].reshape(NB * H * W, C)
    a2d = jnp.maximum(x2d * s1_ref[...] + t1_ref[...], 0.0)
    a = a2d.astype(jnp.bfloat16).reshape(NB, H, W, C)

    zc = jnp.zeros((NB, H, 1, C), jnp.bfloat16)
    aw = jnp.concatenate([zc, a, zc], axis=2)
    zr = jnp.zeros((NB, 1, W + 2, C), jnp.bfloat16)
    ap = jnp.concatenate([zr, aw, zr], axis=1)

    def parity(pr, pc, w_ref):
        taps = [ap[:, pr + ty:pr + ty + H, pc + tx:pc + tx + W, :]
                for ty in range(2) for tx in range(2)]
        im = jnp.concatenate(taps, axis=-1).reshape(NB * H * W, 4 * C)
        return (jnp.dot(im, w_ref[...], preferred_element_type=jnp.float32)
                + b3a_ref[...])

    o00 = parity(0, 0, wee_ref)
    o01 = parity(0, 1, weo_ref)
    o10 = parity(1, 0, woe_ref)
    o11 = parity(1, 1, woo_ref)

    s = (jnp.sum(o00, axis=0, keepdims=True) + jnp.sum(o01, axis=0, keepdims=True)
         + jnp.sum(o10, axis=0, keepdims=True) + jnp.sum(o11, axis=0, keepdims=True))
    ss = (jnp.sum(o00 * o00, axis=0, keepdims=True)
          + jnp.sum(o01 * o01, axis=0, keepdims=True)
          + jnp.sum(o10 * o10, axis=0, keepdims=True)
          + jnp.sum(o11 * o11, axis=0, keepdims=True))
    stats_ref[0, 0:1, :] = s
    stats_ref[0, 1:2, :] = ss

    he = _interleave_rows(o00.astype(jnp.bfloat16).reshape(NB, H, W, C),
                          o10.astype(jnp.bfloat16).reshape(NB, H, W, C))
    ho = _interleave_rows(o01.astype(jnp.bfloat16).reshape(NB, H, W, C),
                          o11.astype(jnp.bfloat16).reshape(NB, H, W, C))
    h_ref[...] = jnp.concatenate([he, ho], axis=-1)


def _stage2_kernel(h_ref, x_ref, s2_ref, t2_ref, w3b_ref,
                   w1_ref, b1_ref, sel_ref, o_ref):
    NB, H2, W, C2 = h_ref.shape
    C = C2 // 2
    H = H2 // 2
    W2 = 2 * W
    Cout = o_ref.shape[-1]

    h = h_ref[...].astype(jnp.float32)
    s2 = s2_ref[...]
    t2 = t2_ref[...]
    ae = jnp.maximum(h[..., :C] * s2 + t2, 0.0).astype(jnp.bfloat16)
    ao = jnp.maximum(h[..., C:] * s2 + t2, 0.0).astype(jnp.bfloat16)

    zc = jnp.zeros((NB, H2, 1, C), jnp.bfloat16)
    ao_m1 = jnp.concatenate([zc, ao[:, :, :W - 1]], axis=2)
    ae_p1 = jnp.concatenate([ae[:, :, 1:], zc], axis=2)
    zr = jnp.zeros((NB, 1, W, C), jnp.bfloat16)

    def hp(t):
        return jnp.concatenate([zr, t, zr], axis=1)

    pae, pao, pao_m1, pae_p1 = hp(ae), hp(ao), hp(ao_m1), hp(ae_p1)

    w3b = w3b_ref[...]
    ev = [src[:, dy:dy + H2] for dy in range(3) for src in (pao_m1, pae, pao)]
    even_im = jnp.concatenate(ev, axis=-1).reshape(NB * H2 * W, 9 * C)
    ye = jnp.dot(even_im, w3b, preferred_element_type=jnp.float32)
    od = [src[:, dy:dy + H2] for dy in range(3) for src in (pae, pao, pae_p1)]
    odd_im = jnp.concatenate(od, axis=-1).reshape(NB * H2 * W, 9 * C)
    yo = jnp.dot(odd_im, w3b, preferred_element_type=jnp.float32)

    x2d = x_ref[...].reshape(NB * H * W, C).astype(jnp.bfloat16)
    sc = (jnp.dot(x2d, w1_ref[...], preferred_element_type=jnp.float32)
          + b1_ref[...]).reshape(NB, H, W, Cout)
    sc2 = _interleave_rows(sc, sc)

    n = NB * H2
    st = jnp.concatenate([ye.reshape(n, W, Cout), yo.reshape(n, W, Cout),
                          sc2.reshape(n, W, Cout)], axis=1)
    sel = jnp.broadcast_to(sel_ref[...], (n, W2, 3 * W))
    out = jnp.einsum("hxw,hwc->hxc", sel, st,
                     preferred_element_type=jnp.float32)
    o_ref[...] = out.reshape(NB, H2, W2, Cout)


def kernel(x, w1x1, b1x1, w3a_flat, b3a, w3b_flat, b3b, g1, be1, g2, be2):
    B, H, W, C = x.shape
    Cout = w1x1.shape[-1]
    H2, W2 = 2 * H, 2 * W
    x = x.astype(jnp.float32)
    NB1 = 16 if B % 16 == 0 else (2 if B % 2 == 0 else 1)
    NB = 8 if B % 8 == 0 else (2 if B % 2 == 0 else 1)

    m1 = jnp.mean(x, axis=(0, 1, 2))
    v1 = jnp.mean(x * x, axis=(0, 1, 2)) - m1 * m1
    inv1 = lax.rsqrt(v1 + _BN_EPS)
    scale1 = (g1 * inv1).reshape(1, C)
    shift1 = (be1 - m1 * g1 * inv1).reshape(1, C)

    w9 = w3a_flat.reshape(3, 3, C, C)
    groups = (((0,), (1, 2)), ((0, 1), (2,)))

    def comb(pr, pc):
        blocks = []
        for ty in range(2):
            for tx in range(2):
                blk = 0.0
                for dy in groups[pr][ty]:
                    for dx in groups[pc][tx]:
                        blk = blk + w9[dy, dx]
                blocks.append(blk)
        return jnp.concatenate(blocks, axis=0).astype(jnp.bfloat16)

    wee, weo, woe, woo = comb(0, 0), comb(0, 1), comb(1, 0), comb(1, 1)

    cparams = pltpu.CompilerParams(
        dimension_semantics=("parallel",),
        vmem_limit_bytes=100 * 1024 * 1024,
    )

    h, stats = pl.pallas_call(
        _stage1_kernel,
        out_shape=(jax.ShapeDtypeStruct((B, H2, W, 2 * C), jnp.bfloat16),
                   jax.ShapeDtypeStruct((B // NB1, 2, C), jnp.float32)),
        grid=(B // NB1,),
        in_specs=[
            pl.BlockSpec((NB1, H, W, C), lambda b: (b, 0, 0, 0)),
            _whole((1, C)), _whole((1, C)),
            _whole((4 * C, C)), _whole((4 * C, C)),
            _whole((4 * C, C)), _whole((4 * C, C)),
            _whole((1, C)),
        ],
        out_specs=(
            pl.BlockSpec((NB1, H2, W, 2 * C), lambda b: (b, 0, 0, 0)),
            pl.BlockSpec((1, 2, C), lambda b: (b, 0, 0)),
        ),
        compiler_params=cparams,
    )(x, scale1, shift1, wee, weo, woe, woo, b3a)

    n2 = jnp.float32(B * H2 * W2)
    tot = jnp.sum(stats, axis=0)
    m2 = tot[0] / n2
    v2 = tot[1] / n2 - m2 * m2
    inv2 = lax.rsqrt(v2 + _BN_EPS)
    scale2 = (g2 * inv2).reshape(1, C)
    shift2 = (be2 - m2 * g2 * inv2).reshape(1, C)

    xi = lax.broadcasted_iota(jnp.int32, (1, W2, 3 * W), 1)
    wi = lax.broadcasted_iota(jnp.int32, (1, W2, 3 * W), 2)
    sel = ((xi == 2 * wi) | (xi == 2 * wi - 2 * W + 1)
           | (xi == 2 * wi - 4 * W) | (xi == 2 * wi - 4 * W + 1)
           ).astype(jnp.float32)

    out = pl.pallas_call(
        _stage2_kernel,
        out_shape=jax.ShapeDtypeStruct((B, H2, W2, Cout), jnp.float32),
        grid=(B // NB,),
        in_specs=[
            pl.BlockSpec((NB, H2, W, 2 * C), lambda b: (b, 0, 0, 0)),
            pl.BlockSpec((NB, H, W, C), lambda b: (b, 0, 0, 0)),
            _whole((1, C)), _whole((1, C)),
            _whole((9 * C, Cout)),
            _whole((C, Cout)), _whole((1, Cout)),
            _whole((1, W2, 3 * W)),
        ],
        out_specs=pl.BlockSpec((NB, H2, W2, Cout), lambda b: (b, 0, 0, 0)),
        compiler_params=cparams,
    )(h, x, scale2, shift2, w3b_flat.astype(jnp.bfloat16),
      w1x1.astype(jnp.bfloat16), b1x1 + b3b, sel)
    return out
